# baseline (speedup 1.0000x reference)
import functools

import jax
import jax.numpy as jnp
from jax import lax
from jax.experimental import pallas as pl
from jax.experimental.pallas import tpu as pltpu
from jax.experimental.pallas import tpu_sc as plsc

_D = 32
_DO = 2 * _D + 2
_GRP = 4
_NC = 2
_NS = 16
_NW = _NC * _NS
_L = 16
_CW = 128
_SW = 64
_LB = 8


@functools.lru_cache(maxsize=None)
def _build(L, B, V_item, V_cat):
    cols_per_w = B // _NW
    cpw = cols_per_w // _CW
    n_lb = L // _LB
    n_subs = cpw * _LB * (_CW // _SW)
    jgroups = _SW // _L
    pgroups = _LB * cols_per_w // _L

    def body(item_idx, cat_idx, amt, ts, item_tab, cat_tab, out,
             ii_b, ci_b, sisc_b, amt_b, ts_b, ts_pm,
             gi_a, gc_a, gi_b2, gc_b2, buf0, buf1,
             sem_ia, sem_ca, sem_ib, sem_cb, sem_o0, sem_o1):
        wid = lax.axis_index("s") * _NC + lax.axis_index("c")
        b0 = wid * cols_per_w
        lanes = lax.iota(jnp.int32, _L)
        bufs = (buf0, buf1)
        sems_o = (sem_o0, sem_o1)
        gsets = ((gi_a, gc_a, sem_ia, sem_ca),
                 (gi_b2, gc_b2, sem_ib, sem_cb))

        def prep_body(i, _):
            row = i >> 5
            g = i & 31
            s = pl.ds(g * _L, _L)
            vi = jnp.clip(ii_b[row, s], 0, V_item - 1)
            vc = jnp.clip(ci_b[row, s], 0, V_cat - 1)
            si = (vi & (_GRP - 1)) << 5
            sc = (vc & (_GRP - 1)) << 5
            sisc_b[row, s] = si | (sc << 8)
            ii_b[row, s] = vi >> 2
            ci_b[row, s] = vc >> 2
            return 0

        def sub_coords(t):
            cb = t >> 4
            lr = (t >> 1) & 7
            q = t & 1
            soff = pl.multiple_of(cb * _CW + q * _SW, _SW)
            return cb, lr, q, soff

        def issue_gather(t, parity):
            _, lr, _, soff = sub_coords(t)
            gi, gc, s_i, s_c = gsets[parity]
            s = pl.ds(soff, _SW)
            cp_i = pltpu.async_copy(item_tab.at[ii_b.at[lr, s]], gi, s_i)
            cp_c = pltpu.async_copy(cat_tab.at[ci_b.at[lr, s]], gc, s_c)
            return cp_i, cp_c

        def wait_gather(parity):
            gi, gc, s_i, s_c = gsets[parity]
            z = pl.ds(0, _SW)
            pltpu.make_async_copy(item_tab.at[ii_b.at[0, z]], gi, s_i).wait()
            pltpu.make_async_copy(cat_tab.at[ci_b.at[0, z]], gc, s_c).wait()

        def assemble(t, b, l0, parity):
            _, lr, q, soff = sub_coords(t)
            gi, gc, _, _ = gsets[parity]
            lrv = jnp.full((_L,), lr, jnp.int32)
            lc = jnp.full((_L,), l0 + lr, jnp.int32)
            lm1 = jnp.maximum(lr - 1, 0)
            ooff = pl.multiple_of(q * _SW, _SW)

            def jg_body(j, _):
                rows = j * _L + lanes
                s = pl.ds(soff + j * _L, _L)
                o = pl.ds(ooff + j * _L, _L)
                v = sisc_b[lr, s]
                si = v & 0xFF
                sc = v >> 8
                for c in range(_D):
                    b[c, lr, o] = plsc.load_gather(gi, [rows, si + c])
                    b[_D + c, lr, o] = plsc.load_gather(gc, [rows, sc + c])
                av = amt_b[lr, s]
                prev = jnp.where(lrv == 0, ts_pm[_LB - 1, s], ts_b[lm1, s])
                d = ts_b[lr, s] - prev
                d = jnp.where(lc == 0, jnp.float32(0), d)
                b[2 * _D, lr, o] = av
                b[2 * _D + 1, lr, o] = d
                return 0

            lax.fori_loop(0, jgroups, jg_body, 0)

        def out_slice(l0, cb):
            boff = pl.multiple_of(b0 + cb * _CW, _CW)
            return out.at[:, pl.ds(l0, _LB), pl.ds(boff, _CW)]

        def lb_body(lb, _):
            l0 = pl.multiple_of(lb * _LB, _LB)
            pltpu.sync_copy(item_idx.at[pl.ds(l0, _LB),
                                        pl.ds(b0, cols_per_w)], ii_b)
            pltpu.sync_copy(cat_idx.at[pl.ds(l0, _LB),
                                       pl.ds(b0, cols_per_w)], ci_b)
            pltpu.sync_copy(amt.at[pl.ds(l0, _LB), pl.ds(b0, cols_per_w)],
                            amt_b)
            pltpu.sync_copy(ts.at[pl.ds(l0, _LB), pl.ds(b0, cols_per_w)],
                            ts_b)
            lpm = pl.multiple_of(jnp.maximum(l0 - _LB, 0), _LB)
            pltpu.sync_copy(ts.at[pl.ds(lpm, _LB), pl.ds(b0, cols_per_w)],
                            ts_pm)
            lax.fori_loop(0, pgroups, prep_body, 0)

            ppb = (_LB * _CW // _SW) // 2
            b = bufs[0]
            issue_gather(0, 0)
            issue_gather(1, 1)

            def pair_body(p, _):
                t0 = p * 2
                t1 = t0 + 1
                wait_gather(0)

                @pl.when((p % ppb == 0) & (p > 0))
                def _():
                    pltpu.make_async_copy(b, out_slice(l0, 0),
                                          sems_o[0]).wait()

                assemble(t0, b, l0, 0)

                @pl.when(p < n_subs // 2 - 1)
                def _():
                    issue_gather(t0 + 2, 0)

                wait_gather(1)
                assemble(t1, b, l0, 1)

                @pl.when(p < n_subs // 2 - 1)
                def _():
                    issue_gather(t1 + 2, 1)

                @pl.when(p % ppb == ppb - 1)
                def _():
                    pltpu.async_copy(b, out_slice(l0, t1 >> 4), sems_o[0])

                return 0

            lax.fori_loop(0, n_subs // 2, pair_body, 0)
            pltpu.make_async_copy(b, out_slice(l0, cpw - 1),
                                  sems_o[0]).wait()
            return 0

        lax.fori_loop(0, n_lb, lb_body, 0)

    return pl.kernel(
        body,
        out_type=jax.ShapeDtypeStruct((_DO, L, B), jnp.float32),
        mesh=plsc.VectorSubcoreMesh(core_axis_name="c", subcore_axis_name="s",
                                    num_cores=_NC, num_subcores=_NS),
        scratch_types=[
            pltpu.VMEM((_LB, B // _NW), jnp.int32),
            pltpu.VMEM((_LB, B // _NW), jnp.int32),
            pltpu.VMEM((_LB, B // _NW), jnp.int32),
            pltpu.VMEM((_LB, B // _NW), jnp.float32),
            pltpu.VMEM((_LB, B // _NW), jnp.float32),
            pltpu.VMEM((_LB, B // _NW), jnp.float32),
            pltpu.VMEM((_SW, _GRP * _D), jnp.float32),
            pltpu.VMEM((_SW, _GRP * _D), jnp.float32),
            pltpu.VMEM((_SW, _GRP * _D), jnp.float32),
            pltpu.VMEM((_SW, _GRP * _D), jnp.float32),
            pltpu.VMEM((_DO, _LB, _CW), jnp.float32),
            pltpu.VMEM((_DO, _LB, _CW), jnp.float32),
            pltpu.SemaphoreType.DMA,
            pltpu.SemaphoreType.DMA,
            pltpu.SemaphoreType.DMA,
            pltpu.SemaphoreType.DMA,
            pltpu.SemaphoreType.DMA,
            pltpu.SemaphoreType.DMA,
        ],
        compiler_params=pltpu.CompilerParams(needs_layout_passes=False),
    )


def kernel(item_ids, cat_ids, amount, timestamp, seq_lens, item_table,
           cat_table):
    del seq_lens
    B, L = item_ids.shape
    ii = item_ids.T.astype(jnp.int32)
    ci = cat_ids.T.astype(jnp.int32)
    am = amount.T.astype(jnp.float32)
    ts = timestamp.T.astype(jnp.float32)
    tab_i = item_table.reshape(item_table.shape[0] // _GRP, _GRP * _D)
    tab_c = cat_table.reshape(cat_table.shape[0] // _GRP, _GRP * _D)
    fn = _build(L, B, item_table.shape[0], cat_table.shape[0])
    out = fn(ii, ci, am, ts, tab_i, tab_c)
    return out.transpose(2, 1, 0)


# speedup vs baseline: 1.1888x; 1.1888x over previous
import functools

import jax
import jax.numpy as jnp
from jax import lax
from jax.experimental import pallas as pl
from jax.experimental.pallas import tpu as pltpu
from jax.experimental.pallas import tpu_sc as plsc

_D = 32
_DO = 2 * _D + 2
_GRP = 4
_NC = 2
_NS = 16
_NW = _NC * _NS
_L = 16
_SC = 8
_H0 = 104


@functools.lru_cache(maxsize=None)
def _build(N, L, V_item, V_cat):
    chunk = L
    h1 = chunk - _H0
    srows = _SC * chunk
    rows_per_w = N // _NW
    n_supers = rows_per_w // srows
    n_groups = -(-chunk // _L)
    prep_groups = srows // _L

    def body(item_idx, cat_idx, amt, ts, item_tab, cat_tab, out,
             ii_s, ci_s, amt_s, ts_s, sisc_s,
             gi_a, gc_a, gi_b, gc_b, buf0, buf1,
             sem_ia, sem_ca, sem_ib, sem_cb, sem_o0, sem_o1):
        wid = lax.axis_index("s") * _NC + lax.axis_index("c")
        lanes = lax.iota(jnp.int32, _L)
        tail_n = chunk - (n_groups - 1) * _L
        col_a = jnp.full((_L,), 2 * _D, jnp.int32)
        col_d = jnp.full((_L,), 2 * _D + 1, jnp.int32)
        bufs = (buf0, buf1)
        sems_o = (sem_o0, sem_o1)
        gsets = ((gi_a, gc_a, sem_ia, sem_ca),
                 (gi_b, gc_b, sem_ib, sem_cb))

        def prep_body(g, _):
            s = pl.ds(g * _L, _L)
            vi = jnp.clip(ii_s[s], 0, V_item - 1)
            vc = jnp.clip(ci_s[s], 0, V_cat - 1)
            si = (vi & (_GRP - 1)) << 5
            sc = (vc & (_GRP - 1)) << 5
            sisc_s[s] = si | (sc << 8)
            ii_s[s] = vi >> 2
            ci_s[s] = vc >> 2
            return 0

        def issue_gather(t):
            k, parity = divmod(t, 2)
            off = k * chunk + (_H0 if parity else 0)
            n = h1 if parity else _H0
            gi, gc, s_i, s_c = gsets[parity]
            cp_i = pltpu.async_copy(
                item_tab.at[ii_s.at[pl.ds(off, n)]], gi, s_i)
            cp_c = pltpu.async_copy(
                cat_tab.at[ci_s.at[pl.ds(off, n)]], gc, s_c)
            return cp_i, cp_c

        def assemble(t, b):
            k, parity = divmod(t, 2)
            off = k * chunk + (_H0 if parity else 0)
            hb = _H0 if parity else 0
            n = h1 if parity else _H0
            gi, gc, _, _ = gsets[parity]

            def row_body(r, _):
                v = sisc_s[pl.ds(off + r, _L)][0]
                oi = v & 0xFF
                oc = v >> 8
                for h in range(2):
                    b[hb + r, pl.ds(h * _L, _L)] = gi[r, pl.ds(oi + h * _L,
                                                               _L)]
                    b[hb + r, pl.ds(_D + h * _L, _L)] = gc[r,
                                                           pl.ds(oc + h * _L,
                                                                 _L)]
                return 0

            lax.fori_loop(0, n, row_body, 0, unroll=2)

        def numeric(k, b):
            off = k * chunk
            for g in range(n_groups):
                o = g * _L + lanes
                msk = (lanes < tail_n) if g == n_groups - 1 else None
                a = ts_s[pl.ds(off + g * _L, _L)]
                pidx = off + o - 1
                if g == 0 and k == 0:
                    pidx = jnp.maximum(pidx, 0)
                d = a - plsc.load_gather(ts_s, [pidx])
                if g == 0:
                    d = jnp.where(lanes == 0, jnp.float32(0), d)
                av = amt_s[pl.ds(off + g * _L, _L)]
                plsc.store_scatter(b, [o, col_a], av, mask=msk)
                plsc.store_scatter(b, [o, col_d], d, mask=msk)

        def super_body(sup, _):
            sbase = wid * rows_per_w + sup * srows
            pltpu.sync_copy(item_idx.at[pl.ds(sbase, srows)],
                            ii_s.at[pl.ds(0, srows)])
            pltpu.sync_copy(cat_idx.at[pl.ds(sbase, srows)],
                            ci_s.at[pl.ds(0, srows)])
            pltpu.sync_copy(amt.at[pl.ds(sbase, srows)],
                            amt_s.at[pl.ds(0, srows)])
            pltpu.sync_copy(ts.at[pl.ds(sbase, srows)],
                            ts_s.at[pl.ds(0, srows)])
            lax.fori_loop(0, prep_groups, prep_body, 0)

            cps = issue_gather(0)
            out_cps = [None, None]
            for t in range(2 * _SC):
                k, parity = divmod(t, 2)
                cb = k % 2
                nxt = issue_gather(t + 1) if t + 1 < 2 * _SC else None
                cps[0].wait()
                cps[1].wait()
                b = bufs[cb]
                if parity == 0 and out_cps[cb] is not None:
                    out_cps[cb].wait()
                assemble(t, b)
                if parity == 1:
                    numeric(k, b)
                    out_cps[cb] = pltpu.async_copy(
                        b, out.at[pl.ds(sbase + k * chunk, chunk)],
                        sems_o[cb])
                cps = nxt
            out_cps[0].wait()
            out_cps[1].wait()
            return 0

        lax.fori_loop(0, n_supers, super_body, 0)

    return pl.kernel(
        body,
        out_type=jax.ShapeDtypeStruct((N, _DO), jnp.float32),
        mesh=plsc.VectorSubcoreMesh(core_axis_name="c", subcore_axis_name="s",
                                    num_cores=_NC, num_subcores=_NS),
        scratch_types=[
            pltpu.VMEM((_SC * L,), jnp.int32),
            pltpu.VMEM((_SC * L,), jnp.int32),
            pltpu.VMEM((_SC * L + _L,), jnp.float32),
            pltpu.VMEM((_SC * L + _L,), jnp.float32),
            pltpu.VMEM((_SC * L + _L,), jnp.int32),
            pltpu.VMEM((_H0, _GRP * _D), jnp.float32),
            pltpu.VMEM((_H0, _GRP * _D), jnp.float32),
            pltpu.VMEM((L - _H0, _GRP * _D), jnp.float32),
            pltpu.VMEM((L - _H0, _GRP * _D), jnp.float32),
            pltpu.VMEM((L, _DO), jnp.float32),
            pltpu.VMEM((L, _DO), jnp.float32),
            pltpu.SemaphoreType.DMA,
            pltpu.SemaphoreType.DMA,
            pltpu.SemaphoreType.DMA,
            pltpu.SemaphoreType.DMA,
            pltpu.SemaphoreType.DMA,
            pltpu.SemaphoreType.DMA,
        ],
        compiler_params=pltpu.CompilerParams(needs_layout_passes=False),
    )


def kernel(item_ids, cat_ids, amount, timestamp, seq_lens, item_table,
           cat_table):
    del seq_lens
    B, L = item_ids.shape
    N = B * L
    ii = item_ids.reshape(N).astype(jnp.int32)
    ci = cat_ids.reshape(N).astype(jnp.int32)
    am = amount.reshape(N).astype(jnp.float32)
    ts = timestamp.reshape(N).astype(jnp.float32)
    tab_i = item_table.reshape(item_table.shape[0] // _GRP, _GRP * _D)
    tab_c = cat_table.reshape(cat_table.shape[0] // _GRP, _GRP * _D)
    fn = _build(N, L, item_table.shape[0], cat_table.shape[0])
    out = fn(ii, ci, am, ts, tab_i, tab_c)
    return out.reshape(B, L, _DO)
